# baseline (speedup 1.0000x reference)
import jax
import jax.numpy as jnp
from jax.experimental import pallas as pl
from jax.experimental.pallas import tpu as pltpu

EPS = 1e-06
D_LAT = 2048
D_EMB = 64
D_IN = D_LAT + D_EMB
TM = 2048


def _mlp_kernel(act_ref, lat_ref, emb_ref, w1_ref, rms_ref, b1_ref, w2_ref,
                b2_ref, out_ref, w1a_s, w1b_s, w2_s):
    del act_ref
    b = pl.program_id(0)
    i = pl.program_id(1)

    @pl.when(jnp.logical_and(b == 0, i == 0))
    def _init():
        w1a_s[...] = (w1_ref[:D_LAT, :] * rms_ref[:D_LAT, :]).astype(jnp.bfloat16)
        w1b_s[...] = (w1_ref[D_LAT:, :] * rms_ref[D_LAT:, :]).astype(jnp.bfloat16)
        w2_s[...] = w2_ref[...].astype(jnp.bfloat16)

    x = lat_ref[0]
    emb = emb_ref[0]
    sumsq = jnp.sum(x * x, axis=-1, keepdims=True) + jnp.sum(emb * emb)
    scale = jax.lax.rsqrt(sumsq * (1.0 / D_IN) + EPS)
    pre = jnp.dot(x.astype(jnp.bfloat16), w1a_s[...],
                  preferred_element_type=jnp.float32)
    ev = jnp.dot(emb.astype(jnp.bfloat16), w1b_s[...],
                 preferred_element_type=jnp.float32)
    h = scale * (pre + ev) + b1_ref[...]
    h = jnp.maximum(h, 0.0).astype(jnp.bfloat16)
    out = jnp.dot(h, w2_s[...], preferred_element_type=jnp.float32)
    out_ref[0] = out + b2_ref[...]


@jax.jit
def kernel(latent, action_type, emb_table, rms_weight, W1, b1, W2, b2):
    B, T, _ = latent.shape
    HID = W1.shape[1]
    MAX_ACT = W2.shape[1]

    act = action_type.astype(jnp.int32)
    emb3 = emb_table.reshape(emb_table.shape[0], 1, D_EMB)
    rms2 = rms_weight.reshape(D_IN, 1)
    b1r = b1.reshape(1, HID)
    b2r = b2.reshape(1, MAX_ACT)

    grid = (B, T // TM)
    grid_spec = pltpu.PrefetchScalarGridSpec(
        num_scalar_prefetch=1,
        grid=grid,
        in_specs=[
            pl.BlockSpec((1, TM, D_LAT), lambda b, i, act: (b, i, 0)),
            pl.BlockSpec((1, 1, D_EMB), lambda b, i, act: (act[b], 0, 0)),
            pl.BlockSpec((D_IN, HID), lambda b, i, act: (0, 0)),
            pl.BlockSpec((D_IN, 1), lambda b, i, act: (0, 0)),
            pl.BlockSpec((1, HID), lambda b, i, act: (0, 0)),
            pl.BlockSpec((HID, MAX_ACT), lambda b, i, act: (0, 0)),
            pl.BlockSpec((1, MAX_ACT), lambda b, i, act: (0, 0)),
        ],
        out_specs=pl.BlockSpec((1, TM, MAX_ACT), lambda b, i, act: (b, i, 0)),
        scratch_shapes=[
            pltpu.VMEM((D_LAT, 512), jnp.bfloat16),
            pltpu.VMEM((D_EMB, 512), jnp.bfloat16),
            pltpu.VMEM((512, 32), jnp.bfloat16),
        ],
    )
    return pl.pallas_call(
        _mlp_kernel,
        grid_spec=grid_spec,
        out_shape=jax.ShapeDtypeStruct((B, T, MAX_ACT), jnp.float32),
        compiler_params=pltpu.CompilerParams(
            dimension_semantics=("arbitrary", "arbitrary"),
        ),
    )(act, latent, emb3, W1, rms2, b1r, W2, b2r)
